# baseline (speedup 1.0000x reference)
import jax
import jax.numpy as jnp
import numpy as np
from jax.experimental import pallas as pl
from jax.experimental.pallas import tpu as pltpu
from jax.experimental.pallas import tpu_sc as plsc

SEQ = 200
BATCH = 6
BPAD = 8
EMBED = 128
HIDDEN = 512
GW = 128
NIDX = SEQ * BPAD
NP1 = 2 * GW
STEPS1 = NP1 // BPAD
STEPS2 = SEQ - STEPS1
NP2 = 12 * GW
CW = 128


def _interleave_gates(w):
    lead = w.shape[:-1]
    w = w.reshape(*lead, 4, HIDDEN // CW, CW)
    w = jnp.swapaxes(w, -3, -2)
    return w.reshape(*lead, 4 * HIDDEN)


def _sc_gather(table, idx2):
    n = idx2.shape[1]
    mesh = plsc.VectorSubcoreMesh(core_axis_name="c", subcore_axis_name="s")

    nw = n // GW
    half = max(nw // 2, 1)

    @pl.kernel(out_type=jax.ShapeDtypeStruct((n, EMBED), table.dtype),
               mesh=mesh)
    def gather_kernel(table_hbm, idx_hbm, out_hbm):
        def body(i_vmem, o_vmem):
            pltpu.sync_copy(table_hbm.at[i_vmem.at[0]], o_vmem)

        pltpu.emit_pipeline(
            body,
            grid=(nw // half, half),
            in_specs=[pl.BlockSpec((1, GW), lambda i, j: (0, i * half + j))],
            out_specs=[pl.BlockSpec((GW, EMBED),
                                    lambda i, j: (i * half + j, 0))],
            core_axis_name=("c", "s"),
            dimension_semantics=(pltpu.PARALLEL, pltpu.PARALLEL),
        )(idx_hbm, out_hbm)

    return gather_kernel(table, idx2)


def _xproj(emb_ref, wih_ref, b2_ref, xproj_ref):
    xproj_ref[...] = (
        jnp.dot(emb_ref[...].astype(jnp.bfloat16), wih_ref[...],
                preferred_element_type=jnp.float32)
        + b2_ref[...]
    )


def _steps(xproj_ref, whh_ref, h, c, nsteps, unroll):
    def step(t, carry):
        h, c = carry
        gates = xproj_ref[pl.ds(t * BPAD, BPAD), :] + jnp.dot(
            h.astype(jnp.bfloat16), whh_ref[...],
            preferred_element_type=jnp.float32,
        )
        hs, cs = [], []
        for k in range(HIDDEN // CW):
            base = 4 * CW * k
            ik = jax.nn.sigmoid(gates[:, base:base + CW])
            fk = jax.nn.sigmoid(gates[:, base + CW:base + 2 * CW])
            gk = jnp.tanh(gates[:, base + 2 * CW:base + 3 * CW])
            ok = jax.nn.sigmoid(gates[:, base + 3 * CW:base + 4 * CW])
            ck = fk * c[:, k * CW:(k + 1) * CW] + ik * gk
            cs.append(ck)
            hs.append(ok * jnp.tanh(ck))
        h = jnp.concatenate(hs, axis=1)
        c = jnp.concatenate(cs, axis=1)
        return (h, c)

    return jax.lax.fori_loop(0, nsteps, step, (h, c), unroll=unroll)


def _lstm1_kernel(emb_ref, wih_ref, whh_ref, b2_ref, hc_ref, xproj_ref):
    _xproj(emb_ref, wih_ref, b2_ref, xproj_ref)
    h0 = jnp.zeros((BPAD, HIDDEN), jnp.float32)
    c0 = jnp.zeros((BPAD, HIDDEN), jnp.float32)
    h, c = _steps(xproj_ref, whh_ref, h0, c0, STEPS1, 16)
    hc_ref[...] = jnp.concatenate([h, c], axis=0)


def _lstm2_kernel(emb_ref, wih_ref, whh_ref, b2_ref, hc_ref, wout_ref,
                  bout_ref, wm_ref, bres_ref, out_ref, xproj_ref):
    _xproj(emb_ref, wih_ref, b2_ref, xproj_ref)
    h, c = _steps(xproj_ref, whh_ref, hc_ref[:BPAD, :], hc_ref[BPAD:, :],
                  STEPS2, 8)

    out = jnp.dot(h, wout_ref[...], preferred_element_type=jnp.float32) \
        + bout_ref[...]
    res = jnp.sum(out[None, :, :] * wm_ref[...], axis=(1, 2))
    out_ref[...] = res.reshape(1, 2) + bres_ref[...]


def kernel(input_data, table, W_ih, W_hh, b_ih, b_hh, W_out, b_out, W_res, b_res):
    idx = jnp.pad(input_data, ((0, 0), (0, BPAD - BATCH)))
    flat_idx = idx.reshape(-1)
    idx1 = flat_idx[:NP1].reshape(1, NP1)
    idx2 = jnp.pad(flat_idx[NP1:], (0, NP1 + NP2 - NIDX)).reshape(1, NP2)
    emb1 = _sc_gather(table, idx1)
    emb2 = _sc_gather(table, idx2)

    b2 = _interleave_gates(b_ih + b_hh).reshape(1, 4 * HIDDEN)
    wih = _interleave_gates(W_ih.T).astype(jnp.bfloat16)
    whh = _interleave_gates(W_hh.T).astype(jnp.bfloat16)
    Wm = jnp.pad(W_res.reshape(2, BATCH, 2), ((0, 0), (0, BPAD - BATCH), (0, 0)))

    hc = pl.pallas_call(
        _lstm1_kernel,
        out_shape=jax.ShapeDtypeStruct((2 * BPAD, HIDDEN), jnp.float32),
        scratch_shapes=[pltpu.VMEM((NP1, 4 * HIDDEN), jnp.float32)],
    )(emb1, wih, whh, b2)

    res = pl.pallas_call(
        _lstm2_kernel,
        out_shape=jax.ShapeDtypeStruct((1, 2), jnp.float32),
        scratch_shapes=[pltpu.VMEM((NP2, 4 * HIDDEN), jnp.float32)],
    )(emb2, wih, whh, b2, hc, W_out.T, b_out.reshape(1, 2), Wm,
      b_res.reshape(1, 2))
    return res


# speedup vs baseline: 1.0693x; 1.0693x over previous
import jax
import jax.numpy as jnp
import numpy as np
from jax.experimental import pallas as pl
from jax.experimental.pallas import tpu as pltpu
from jax.experimental.pallas import tpu_sc as plsc

SEQ = 200
BATCH = 6
BPAD = 8
EMBED = 128
HIDDEN = 512
GW = 128
NIDX = SEQ * BPAD
NP1 = 7 * GW
STEPS1 = NP1 // BPAD
STEPS2 = SEQ - STEPS1
NP2 = 6 * GW
CW = 128


def _interleave_gates(w):
    lead = w.shape[:-1]
    w = w.reshape(*lead, 4, HIDDEN // CW, CW)
    w = jnp.swapaxes(w, -3, -2)
    return w.reshape(*lead, 4 * HIDDEN)


def _sc_gather(table, idx2):
    n = idx2.shape[1]
    mesh = plsc.VectorSubcoreMesh(core_axis_name="c", subcore_axis_name="s")

    nw = n // GW
    half = max(nw // 2, 1)

    @pl.kernel(out_type=jax.ShapeDtypeStruct((n, EMBED), table.dtype),
               mesh=mesh)
    def gather_kernel(table_hbm, idx_hbm, out_hbm):
        def body(i_vmem, o_vmem):
            pltpu.sync_copy(table_hbm.at[i_vmem.at[0]], o_vmem)

        pltpu.emit_pipeline(
            body,
            grid=(nw // half, half),
            in_specs=[pl.BlockSpec((1, GW), lambda i, j: (0, i * half + j))],
            out_specs=[pl.BlockSpec((GW, EMBED),
                                    lambda i, j: (i * half + j, 0))],
            core_axis_name=("c", "s"),
            dimension_semantics=(pltpu.PARALLEL, pltpu.PARALLEL),
        )(idx_hbm, out_hbm)

    return gather_kernel(table, idx2)


def _xproj(emb_ref, wih_ref, b2_ref, xproj_ref):
    xproj_ref[...] = (
        jnp.dot(emb_ref[...].astype(jnp.bfloat16), wih_ref[...],
                preferred_element_type=jnp.float32)
        + b2_ref[...]
    )


def _steps(xproj_ref, whh_ref, h, c, nsteps, unroll):
    def step(t, carry):
        h, c = carry
        gates = xproj_ref[pl.ds(t * BPAD, BPAD), :] + jnp.dot(
            h.astype(jnp.bfloat16), whh_ref[...],
            preferred_element_type=jnp.float32,
        )
        hs, cs = [], []
        for k in range(HIDDEN // CW):
            base = 4 * CW * k
            ik = jax.nn.sigmoid(gates[:, base:base + CW])
            fk = jax.nn.sigmoid(gates[:, base + CW:base + 2 * CW])
            gk = jnp.tanh(gates[:, base + 2 * CW:base + 3 * CW])
            ok = jax.nn.sigmoid(gates[:, base + 3 * CW:base + 4 * CW])
            ck = fk * c[:, k * CW:(k + 1) * CW] + ik * gk
            cs.append(ck)
            hs.append(ok * jnp.tanh(ck))
        h = jnp.concatenate(hs, axis=1)
        c = jnp.concatenate(cs, axis=1)
        return (h, c)

    return jax.lax.fori_loop(0, nsteps, step, (h, c), unroll=unroll)


def _lstm1_kernel(emb_ref, wih_ref, whh_ref, b2_ref, hc_ref, xproj_ref):
    _xproj(emb_ref, wih_ref, b2_ref, xproj_ref)
    h0 = jnp.zeros((BPAD, HIDDEN), jnp.float32)
    c0 = jnp.zeros((BPAD, HIDDEN), jnp.float32)
    h, c = _steps(xproj_ref, whh_ref, h0, c0, STEPS1, 16)
    hc_ref[...] = jnp.concatenate([h, c], axis=0)


def _lstm2_kernel(emb_ref, wih_ref, whh_ref, b2_ref, hc_ref, wout_ref,
                  bout_ref, wm_ref, bres_ref, out_ref, xproj_ref):
    _xproj(emb_ref, wih_ref, b2_ref, xproj_ref)
    h, c = _steps(xproj_ref, whh_ref, hc_ref[:BPAD, :], hc_ref[BPAD:, :],
                  STEPS2, 8)

    out = jnp.dot(h, wout_ref[...], preferred_element_type=jnp.float32) \
        + bout_ref[...]
    res = jnp.sum(out[None, :, :] * wm_ref[...], axis=(1, 2))
    out_ref[...] = res.reshape(1, 2) + bres_ref[...]


def kernel(input_data, table, W_ih, W_hh, b_ih, b_hh, W_out, b_out, W_res, b_res):
    idx = jnp.pad(input_data, ((0, 0), (0, BPAD - BATCH)))
    flat_idx = idx.reshape(-1)
    idx1 = flat_idx[:NP1].reshape(1, NP1)
    idx2 = jnp.pad(flat_idx[NP1:], (0, NP1 + NP2 - NIDX)).reshape(1, NP2)
    emb1 = _sc_gather(table, idx1)
    emb2 = _sc_gather(table, idx2)

    b2 = _interleave_gates(b_ih + b_hh).reshape(1, 4 * HIDDEN)
    wih = _interleave_gates(W_ih.T).astype(jnp.bfloat16)
    whh = _interleave_gates(W_hh.T).astype(jnp.bfloat16)
    Wm = jnp.pad(W_res.reshape(2, BATCH, 2), ((0, 0), (0, BPAD - BATCH), (0, 0)))

    hc = pl.pallas_call(
        _lstm1_kernel,
        out_shape=jax.ShapeDtypeStruct((2 * BPAD, HIDDEN), jnp.float32),
        scratch_shapes=[pltpu.VMEM((NP1, 4 * HIDDEN), jnp.float32)],
    )(emb1, wih, whh, b2)

    res = pl.pallas_call(
        _lstm2_kernel,
        out_shape=jax.ShapeDtypeStruct((1, 2), jnp.float32),
        scratch_shapes=[pltpu.VMEM((NP2, 4 * HIDDEN), jnp.float32)],
    )(emb2, wih, whh, b2, hc, W_out.T, b_out.reshape(1, 2), Wm,
      b_res.reshape(1, 2))
    return res
